# baseline (speedup 1.0000x reference)
import functools

import jax
import jax.numpy as jnp
from jax import lax
from jax.experimental import pallas as pl
from jax.experimental.pallas import tpu as pltpu
from jax.experimental.pallas import tpu_sc as plsc


def _make_gather(batch: int, hist: int, dim: int):
    n_workers = 32
    hb = 4
    b_lo = 128
    h_hi, h_lo = hist // 8, 8
    d_hi, d_lo = dim // 8, 8
    n_rounds = hist // hb
    rows = hb * b_lo
    mesh = plsc.VectorSubcoreMesh(core_axis_name="c", subcore_axis_name="s")

    @functools.partial(
        pl.kernel,
        mesh=mesh,
        out_type=jax.ShapeDtypeStruct((hist, d_hi, n_workers, d_lo, b_lo), jnp.float32),
        scratch_types=[
            pltpu.VMEM((h_hi, h_lo, b_lo), jnp.int32),
            pltpu.VMEM((rows, dim), jnp.float32),
            pltpu.VMEM((rows, dim), jnp.float32),
            pltpu.VMEM((hb, d_hi, d_lo, b_lo), jnp.float32),
            pltpu.VMEM((hb, d_hi, d_lo, b_lo), jnp.float32),
            pltpu.SemaphoreType.DMA,
            pltpu.SemaphoreType.DMA,
            pltpu.SemaphoreType.DMA,
        ],
        compiler_params=pltpu.CompilerParams(
            use_tc_tiling_on_sc=False, needs_layout_passes=False
        ),
    )
    def gather(tab_hbm, ids6_hbm, out6_hbm, idx_v, gbuf_a, gbuf_b, obuf_a, obuf_b,
               gsem, ssem_a, ssem_b):
        wid = lax.axis_index("s") * 2 + lax.axis_index("c")
        pltpu.sync_copy(ids6_hbm.at[:, wid], idx_v)

        iota = lax.iota(jnp.int32, 16)

        def issue_gathers(r, gbuf):
            for hl in range(hb):
                h = r * hb + hl
                pltpu.async_copy(
                    tab_hbm.at[idx_v.at[h // h_lo, h % h_lo]],
                    gbuf.at[pl.ds(hl * b_lo, b_lo)],
                    gsem,
                )

        def drain_gathers(gbuf):
            pltpu.make_async_copy(tab_hbm.at[pl.ds(0, rows)], gbuf, gsem).wait()

        def extract(gbuf, obuf):
            def body(j, carry):
                hl = j // 8
                bg = j % 8
                slotv = iota + hl * b_lo + bg * 16
                for d in range(dim):
                    v = plsc.load_gather(gbuf, [slotv, jnp.full((16,), d, jnp.int32)])
                    obuf[hl, d // d_lo, d % d_lo, pl.ds(bg * 16, 16)] = v
                return carry

            lax.fori_loop(0, hb * (b_lo // 16), body, 0)

        def start_store(r, obuf, sem):
            pltpu.async_copy(obuf, out6_hbm.at[pl.ds(r * hb, hb), :, wid], sem)

        def wait_store(obuf, sem):
            pltpu.make_async_copy(obuf, out6_hbm.at[pl.ds(0, hb), :, wid], sem).wait()

        issue_gathers(0, gbuf_a)

        def body(gg, carry):
            r0 = gg * 2

            @pl.when(gg > 0)
            def _():
                wait_store(obuf_b, ssem_b)

            issue_gathers(r0 + 1, gbuf_b)
            drain_gathers(gbuf_a)

            @pl.when(gg > 0)
            def _():
                wait_store(obuf_a, ssem_a)

            extract(gbuf_a, obuf_a)
            start_store(r0, obuf_a, ssem_a)

            @pl.when(gg < n_rounds // 2 - 1)
            def _():
                issue_gathers(r0 + 2, gbuf_a)

            drain_gathers(gbuf_b)
            extract(gbuf_b, obuf_b)
            start_store(r0 + 1, obuf_b, ssem_b)
            return carry

        lax.fori_loop(0, n_rounds // 2, body, 0)
        wait_store(obuf_a, ssem_a)
        wait_store(obuf_b, ssem_b)

    return gather


def kernel(input_ids, table):
    batch, hist = input_ids.shape
    vocab, dim = table.shape
    assert batch % (32 * 128) == 0 and hist % 8 == 0 and dim % 8 == 0

    ids6 = (
        input_ids.astype(jnp.int32)
        .reshape(32, 128, hist // 8, 8)
        .transpose(2, 0, 3, 1)
    )
    gather = _make_gather(batch, hist, dim)
    out6 = gather(table, ids6)
    return out6.transpose(2, 4, 0, 1, 3).reshape(batch, hist, dim)


# speedup vs baseline: 1.7826x; 1.7826x over previous
import functools

import jax
import jax.numpy as jnp
from jax import lax
from jax.experimental import pallas as pl
from jax.experimental.pallas import tpu as pltpu
from jax.experimental.pallas import tpu_sc as plsc


def _make_gather(batch: int, hist: int, dim: int):
    n_workers = 32
    hb = 4
    b_lo = 128
    h_hi, h_lo = hist // 8, 8
    d_hi, d_lo = dim // 8, 8
    n_rounds = hist // hb
    rows = hb * b_lo
    bp = b_lo + 1
    mesh = plsc.VectorSubcoreMesh(core_axis_name="c", subcore_axis_name="s")

    @functools.partial(
        pl.kernel,
        mesh=mesh,
        out_type=jax.ShapeDtypeStruct((hist, d_hi, n_workers, d_lo, b_lo), jnp.float32),
        scratch_types=[
            pltpu.VMEM((h_hi, h_lo, b_lo), jnp.int32),
            pltpu.VMEM((rows, dim), jnp.float32),
            pltpu.VMEM((rows, dim), jnp.float32),
            pltpu.VMEM((hb, d_hi, d_lo, bp), jnp.float32),
            pltpu.VMEM((hb, d_hi, d_lo, bp), jnp.float32),
            pltpu.SemaphoreType.DMA,
            pltpu.SemaphoreType.DMA,
            pltpu.SemaphoreType.DMA,
        ],
        compiler_params=pltpu.CompilerParams(
            use_tc_tiling_on_sc=False, needs_layout_passes=False
        ),
    )
    def gather(tab_hbm, ids6_hbm, out6_hbm, idx_v, gbuf_a, gbuf_b, obuf_a, obuf_b,
               gsem, ssem_a, ssem_b):
        wid = lax.axis_index("s") * 2 + lax.axis_index("c")
        pltpu.sync_copy(ids6_hbm.at[:, wid], idx_v)

        iota = lax.iota(jnp.int32, 16)
        dhi_lo = iota // d_lo
        dlo_lo = iota % d_lo
        dhi_hi = (iota + 16) // d_lo
        dlo_hi = (iota + 16) % d_lo

        def issue_gathers(r, gbuf):
            for hl in range(hb):
                h = r * hb + hl
                pltpu.async_copy(
                    tab_hbm.at[idx_v.at[h // h_lo, h % h_lo]],
                    gbuf.at[pl.ds(hl * b_lo, b_lo)],
                    gsem,
                )

        def drain_gathers(gbuf):
            pltpu.make_async_copy(tab_hbm.at[pl.ds(0, rows)], gbuf, gsem).wait()

        def extract(gbuf, obuf):
            def body(j, carry):
                hlv = jnp.full((16,), j // b_lo, jnp.int32)
                blov = jnp.full((16,), j % b_lo, jnp.int32)
                va = gbuf[j, pl.ds(0, 16)]
                vb = gbuf[j, pl.ds(16, 16)]
                plsc.store_scatter(obuf, [hlv, dhi_lo, dlo_lo, blov], va)
                plsc.store_scatter(obuf, [hlv, dhi_hi, dlo_hi, blov], vb)
                return carry

            lax.fori_loop(0, rows, body, 0)

        def start_store(r, obuf, sem):
            pltpu.async_copy(
                obuf.at[:, :, :, pl.ds(0, b_lo)],
                out6_hbm.at[pl.ds(r * hb, hb), :, wid],
                sem,
            )

        def wait_store(obuf, sem):
            pltpu.make_async_copy(
                obuf.at[:, :, :, pl.ds(0, b_lo)],
                out6_hbm.at[pl.ds(0, hb), :, wid],
                sem,
            ).wait()

        issue_gathers(0, gbuf_a)

        def body(gg, carry):
            r0 = gg * 2

            @pl.when(gg > 0)
            def _():
                wait_store(obuf_b, ssem_b)

            issue_gathers(r0 + 1, gbuf_b)
            drain_gathers(gbuf_a)

            @pl.when(gg > 0)
            def _():
                wait_store(obuf_a, ssem_a)

            extract(gbuf_a, obuf_a)
            start_store(r0, obuf_a, ssem_a)

            @pl.when(gg < n_rounds // 2 - 1)
            def _():
                issue_gathers(r0 + 2, gbuf_a)

            drain_gathers(gbuf_b)
            extract(gbuf_b, obuf_b)
            start_store(r0 + 1, obuf_b, ssem_b)
            return carry

        lax.fori_loop(0, n_rounds // 2, body, 0)
        wait_store(obuf_a, ssem_a)
        wait_store(obuf_b, ssem_b)

    return gather


def kernel(input_ids, table):
    batch, hist = input_ids.shape
    vocab, dim = table.shape
    assert batch % (32 * 128) == 0 and hist % 8 == 0 and dim == 32

    ids6 = (
        input_ids.astype(jnp.int32)
        .reshape(32, 128, hist // 8, 8)
        .transpose(2, 0, 3, 1)
    )
    gather = _make_gather(batch, hist, dim)
    out6 = gather(table, ids6)
    return out6.transpose(2, 4, 0, 1, 3).reshape(batch, hist, dim)
